# initial kernel scaffold (unmeasured)
import jax
import jax.numpy as jnp
from jax import lax
from jax.experimental import pallas as pl
from jax.experimental.pallas import tpu as pltpu

N_DEV = 16
SQ = 256
D = 1024
HEADS = 8
DH = 128
CH = SQ // N_DEV
SCALE = 0.08838834764831843


def kernel(x, Wq, Wo, Wk, Wv):
    def body(x_ref, wq_ref, wo_ref, wk_ref, wv_ref, out_ref,
             comm_rs, comm_ag, rs_send, rs_recv, ag_send, ag_recv):
        my = lax.axis_index("i")
        right = lax.rem(my + 1, N_DEV)

        xb = x_ref[0, :, :].astype(jnp.bfloat16)
        q = lax.dot_general(
            xb, wq_ref[:, :].astype(jnp.bfloat16),
            (((1,), (0,)), ((), ())), preferred_element_type=jnp.float32)
        k = lax.dot_general(
            xb, wk_ref[:, :].astype(jnp.bfloat16),
            (((1,), (0,)), ((), ())), preferred_element_type=jnp.float32)
        v = lax.dot_general(
            xb, wv_ref[:, :].astype(jnp.bfloat16),
            (((1,), (0,)), ((), ())), preferred_element_type=jnp.float32)

        partial = jnp.zeros((SQ, D), dtype=jnp.float32)
        for h in range(HEADS):
            sl = slice(h * DH, (h + 1) * DH)
            qh = (q[:, sl] * SCALE).astype(jnp.bfloat16)
            kh = k[:, sl].astype(jnp.bfloat16)
            vh = v[:, sl].astype(jnp.bfloat16)
            s = lax.dot_general(qh, kh, (((1,), (1,)), ((), ())),
                                preferred_element_type=jnp.float32)
            m = jnp.max(s, axis=1, keepdims=True)
            p = jnp.exp(s - m)
            l = jnp.sum(p, axis=1, keepdims=True)
            o = lax.dot_general(p.astype(jnp.bfloat16), vh,
                                (((1,), (0,)), ((), ())),
                                preferred_element_type=jnp.float32)
            o = o / l
            partial = partial + lax.dot_general(
                o.astype(jnp.bfloat16),
                wo_ref[sl, :].astype(jnp.bfloat16),
                (((1,), (0,)), ((), ())), preferred_element_type=jnp.float32)
        out_ref[0, :, :] = partial

        for s in range(N_DEV - 1):
            c_send = lax.rem(my - s + N_DEV, N_DEV)
            c_recv = lax.rem(my - s - 1 + 2 * N_DEV, N_DEV)
            rdma = pltpu.make_async_remote_copy(
                src_ref=out_ref.at[0, pl.ds(c_send * CH, CH), :],
                dst_ref=comm_rs.at[s],
                send_sem=rs_send.at[s],
                recv_sem=rs_recv.at[s],
                device_id=(right,),
                device_id_type=pl.DeviceIdType.MESH,
            )
            rdma.start()
            rdma.wait()
            row = c_recv * CH
            out_ref[0, pl.ds(row, CH), :] = (
                out_ref[0, pl.ds(row, CH), :] + comm_rs[s])

        for s in range(N_DEV - 1):
            c_send = lax.rem(my + 1 - s + 2 * N_DEV, N_DEV)
            c_recv = lax.rem(my - s + 2 * N_DEV, N_DEV)
            rdma = pltpu.make_async_remote_copy(
                src_ref=out_ref.at[0, pl.ds(c_send * CH, CH), :],
                dst_ref=comm_ag.at[s],
                send_sem=ag_send.at[s],
                recv_sem=ag_recv.at[s],
                device_id=(right,),
                device_id_type=pl.DeviceIdType.MESH,
            )
            rdma.start()
            rdma.wait()
            out_ref[0, pl.ds(c_recv * CH, CH), :] = comm_ag[s]

    return pl.pallas_call(
        body,
        out_shape=jax.ShapeDtypeStruct((1, SQ, D), jnp.float32),
        in_specs=[pl.BlockSpec(memory_space=pltpu.VMEM)] * 5,
        out_specs=pl.BlockSpec(memory_space=pltpu.VMEM),
        scratch_shapes=[
            pltpu.VMEM((N_DEV - 1, CH, D), jnp.float32),
            pltpu.VMEM((N_DEV - 1, CH, D), jnp.float32),
            pltpu.SemaphoreType.DMA((N_DEV - 1,)),
            pltpu.SemaphoreType.DMA((N_DEV - 1,)),
            pltpu.SemaphoreType.DMA((N_DEV - 1,)),
            pltpu.SemaphoreType.DMA((N_DEV - 1,)),
        ],
        compiler_params=pltpu.CompilerParams(collective_id=0),
    )(x, Wq, Wo, Wk, Wv)


# baseline (device time: 103160 ns/iter reference)
import jax
import jax.numpy as jnp
from jax import lax
from jax.experimental import pallas as pl
from jax.experimental.pallas import tpu as pltpu

N_DEV = 16
SQ = 256
D = 1024
HEADS = 8
DH = 128
CH = SQ // N_DEV
SCALE = 0.08838834764831843


def kernel(x, Wq, Wo, Wk, Wv):
    def body(x_ref, wq_ref, wo_ref, wk_ref, wv_ref, out_ref,
             comm_rs, comm_ag, rs_send, rs_recv, ag_send, ag_recv):
        my = lax.axis_index("i")
        right = lax.rem(my + 1, N_DEV)

        xb = x_ref[0, :, :].astype(jnp.bfloat16)
        q = lax.dot_general(
            xb, wq_ref[:, :].astype(jnp.bfloat16),
            (((1,), (0,)), ((), ())), preferred_element_type=jnp.float32)
        k = lax.dot_general(
            xb, wk_ref[:, :].astype(jnp.bfloat16),
            (((1,), (0,)), ((), ())), preferred_element_type=jnp.float32)
        v = lax.dot_general(
            xb, wv_ref[:, :].astype(jnp.bfloat16),
            (((1,), (0,)), ((), ())), preferred_element_type=jnp.float32)

        partial = jnp.zeros((SQ, D), dtype=jnp.float32)
        for h in range(HEADS):
            sl = slice(h * DH, (h + 1) * DH)
            qh = (q[:, sl] * SCALE).astype(jnp.bfloat16)
            kh = k[:, sl].astype(jnp.bfloat16)
            vh = v[:, sl].astype(jnp.bfloat16)
            s = lax.dot_general(qh, kh, (((1,), (1,)), ((), ())),
                                preferred_element_type=jnp.float32)
            m = jnp.max(s, axis=1, keepdims=True)
            p = jnp.exp(s - m)
            l = jnp.sum(p, axis=1, keepdims=True)
            o = lax.dot_general(p.astype(jnp.bfloat16), vh,
                                (((1,), (0,)), ((), ())),
                                preferred_element_type=jnp.float32)
            o = o / l
            partial = partial + lax.dot_general(
                o.astype(jnp.bfloat16),
                wo_ref[sl, :].astype(jnp.bfloat16),
                (((1,), (0,)), ((), ())), preferred_element_type=jnp.float32)
        out_ref[0, :, :] = partial

        for s in range(N_DEV - 1):
            c_send = lax.rem(my - s + N_DEV, N_DEV)
            c_recv = lax.rem(my - s - 1 + 2 * N_DEV, N_DEV)
            rdma = pltpu.make_async_remote_copy(
                src_ref=out_ref.at[0, pl.ds(c_send * CH, CH), :],
                dst_ref=comm_rs.at[s],
                send_sem=rs_send.at[s],
                recv_sem=rs_recv.at[s],
                device_id=(right,),
                device_id_type=pl.DeviceIdType.MESH,
            )
            rdma.start()
            rdma.wait()
            row = c_recv * CH
            out_ref[0, pl.ds(row, CH), :] = (
                out_ref[0, pl.ds(row, CH), :] + comm_rs[s])

        for s in range(N_DEV - 1):
            c_send = lax.rem(my + 1 - s + 2 * N_DEV, N_DEV)
            c_recv = lax.rem(my - s + 2 * N_DEV, N_DEV)
            rdma = pltpu.make_async_remote_copy(
                src_ref=out_ref.at[0, pl.ds(c_send * CH, CH), :],
                dst_ref=comm_ag.at[s],
                send_sem=ag_send.at[s],
                recv_sem=ag_recv.at[s],
                device_id=(right,),
                device_id_type=pl.DeviceIdType.MESH,
            )
            rdma.start()
            rdma.wait()
            out_ref[0, pl.ds(c_recv * CH, CH), :] = comm_ag[s]

    return pl.pallas_call(
        body,
        out_shape=jax.ShapeDtypeStruct((1, SQ, D), jnp.float32),
        in_specs=[pl.BlockSpec(memory_space=pltpu.VMEM)] * 5,
        out_specs=pl.BlockSpec(memory_space=pltpu.VMEM),
        scratch_shapes=[
            pltpu.VMEM((N_DEV - 1, CH, D), jnp.float32),
            pltpu.VMEM((N_DEV - 1, CH, D), jnp.float32),
            pltpu.SemaphoreType.DMA((N_DEV - 1,)),
            pltpu.SemaphoreType.DMA((N_DEV - 1,)),
            pltpu.SemaphoreType.DMA((N_DEV - 1,)),
            pltpu.SemaphoreType.DMA((N_DEV - 1,)),
        ],
    )(x, Wq, Wo, Wk, Wv)


# device time: 49351 ns/iter; 2.0903x vs baseline; 2.0903x over previous
import jax
import jax.numpy as jnp
from jax import lax
from jax.experimental import pallas as pl
from jax.experimental.pallas import tpu as pltpu

N_DEV = 16
SQ = 256
D = 1024
HEADS = 8
DH = 128
CH = SQ // N_DEV
SCALE = 0.08838834764831843


def kernel(x, Wq, Wo, Wk, Wv):
    def body(x_ref, wq_ref, wo_ref, wk_ref, wv_ref, out_ref,
             comm_rs, rs_send, rs_recv, ag_send, ag_recv):
        my = lax.axis_index("i")

        xb = x_ref[0, :, :].astype(jnp.bfloat16)
        q = lax.dot_general(
            xb, wq_ref[:, :].astype(jnp.bfloat16),
            (((1,), (0,)), ((), ())), preferred_element_type=jnp.float32)
        k = lax.dot_general(
            xb, wk_ref[:, :].astype(jnp.bfloat16),
            (((1,), (0,)), ((), ())), preferred_element_type=jnp.float32)
        v = lax.dot_general(
            xb, wv_ref[:, :].astype(jnp.bfloat16),
            (((1,), (0,)), ((), ())), preferred_element_type=jnp.float32)

        partial = jnp.zeros((SQ, D), dtype=jnp.float32)
        for h in range(HEADS):
            sl = slice(h * DH, (h + 1) * DH)
            qh = (q[:, sl] * SCALE).astype(jnp.bfloat16)
            kh = k[:, sl].astype(jnp.bfloat16)
            vh = v[:, sl].astype(jnp.bfloat16)
            s = lax.dot_general(qh, kh, (((1,), (1,)), ((), ())),
                                preferred_element_type=jnp.float32)
            m = jnp.max(s, axis=1, keepdims=True)
            p = jnp.exp(s - m)
            l = jnp.sum(p, axis=1, keepdims=True)
            o = lax.dot_general(p.astype(jnp.bfloat16), vh,
                                (((1,), (0,)), ((), ())),
                                preferred_element_type=jnp.float32)
            o = o / l
            partial = partial + lax.dot_general(
                o.astype(jnp.bfloat16),
                wo_ref[sl, :].astype(jnp.bfloat16),
                (((1,), (0,)), ((), ())), preferred_element_type=jnp.float32)
        out_ref[0, :, :] = partial

        rs_rdmas = []
        for o in range(1, N_DEV):
            tgt = lax.rem(my + o, N_DEV)
            slot = N_DEV - 1 - o
            rdma = pltpu.make_async_remote_copy(
                src_ref=out_ref.at[0, pl.ds(tgt * CH, CH), :],
                dst_ref=comm_rs.at[slot],
                send_sem=rs_send.at[o - 1],
                recv_sem=rs_recv.at[slot],
                device_id=(tgt,),
                device_id_type=pl.DeviceIdType.MESH,
            )
            rdma.start()
            rs_rdmas.append(rdma)
        for r in rs_rdmas:
            r.wait_recv()
        acc = out_ref[0, pl.ds(my * CH, CH), :]
        for s in range(N_DEV - 1):
            acc = acc + comm_rs[s]
        out_ref[0, pl.ds(my * CH, CH), :] = acc
        for r in rs_rdmas:
            r.wait_send()

        ag_rdmas = []
        for o in range(1, N_DEV):
            tgt = lax.rem(my + o, N_DEV)
            rdma = pltpu.make_async_remote_copy(
                src_ref=out_ref.at[0, pl.ds(my * CH, CH), :],
                dst_ref=out_ref.at[0, pl.ds(my * CH, CH), :],
                send_sem=ag_send.at[o - 1],
                recv_sem=ag_recv.at[o - 1],
                device_id=(tgt,),
                device_id_type=pl.DeviceIdType.MESH,
            )
            rdma.start()
            ag_rdmas.append(rdma)
        for r in ag_rdmas:
            r.wait_recv()
        for r in ag_rdmas:
            r.wait_send()

    return pl.pallas_call(
        body,
        out_shape=jax.ShapeDtypeStruct((1, SQ, D), jnp.float32),
        in_specs=[pl.BlockSpec(memory_space=pltpu.VMEM)] * 5,
        out_specs=pl.BlockSpec(memory_space=pltpu.VMEM),
        scratch_shapes=[
            pltpu.VMEM((N_DEV - 1, CH, D), jnp.float32),
            pltpu.SemaphoreType.DMA((N_DEV - 1,)),
            pltpu.SemaphoreType.DMA((N_DEV - 1,)),
            pltpu.SemaphoreType.DMA((N_DEV - 1,)),
            pltpu.SemaphoreType.DMA((N_DEV - 1,)),
        ],
    )(x, Wq, Wo, Wk, Wv)


# device time: 40422 ns/iter; 2.5521x vs baseline; 1.2209x over previous
import jax
import jax.numpy as jnp
from jax import lax
from jax.experimental import pallas as pl
from jax.experimental.pallas import tpu as pltpu

N_DEV = 16
SQ = 256
D = 1024
HEADS = 8
DH = 128
CH = SQ // N_DEV
SCALE = 0.08838834764831843


def kernel(x, Wq, Wo, Wk, Wv):
    def body(x_ref, wq_ref, wo_ref, wk_ref, wv_ref, out_ref,
             pb_ref, comm_rs, ag_src, ag_buf,
             rs_send, rs_recv, ag_send, ag_recv):
        my = lax.axis_index("i")

        xb = x_ref[0, :, :].astype(jnp.bfloat16)
        q = lax.dot_general(
            xb, wq_ref[:, :].astype(jnp.bfloat16),
            (((1,), (0,)), ((), ())), preferred_element_type=jnp.float32)
        k = lax.dot_general(
            xb, wk_ref[:, :].astype(jnp.bfloat16),
            (((1,), (0,)), ((), ())), preferred_element_type=jnp.float32)
        v = lax.dot_general(
            xb, wv_ref[:, :].astype(jnp.bfloat16),
            (((1,), (0,)), ((), ())), preferred_element_type=jnp.float32)

        partial = jnp.zeros((SQ, D), dtype=jnp.float32)
        for h in range(HEADS):
            sl = slice(h * DH, (h + 1) * DH)
            qh = (q[:, sl] * SCALE).astype(jnp.bfloat16)
            kh = k[:, sl].astype(jnp.bfloat16)
            vh = v[:, sl].astype(jnp.bfloat16)
            s = lax.dot_general(qh, kh, (((1,), (1,)), ((), ())),
                                preferred_element_type=jnp.float32)
            m = jnp.max(s, axis=1, keepdims=True)
            p = jnp.exp(s - m)
            l = jnp.sum(p, axis=1, keepdims=True)
            o = lax.dot_general(p.astype(jnp.bfloat16), vh,
                                (((1,), (0,)), ((), ())),
                                preferred_element_type=jnp.float32)
            o = o / l
            partial = partial + lax.dot_general(
                o.astype(jnp.bfloat16),
                wo_ref[sl, :].astype(jnp.bfloat16),
                (((1,), (0,)), ((), ())), preferred_element_type=jnp.float32)
        out_ref[0, :, :] = partial
        pb_ref[:, :] = partial.astype(jnp.bfloat16)

        rs_rdmas = []
        for o in range(1, N_DEV):
            tgt = lax.rem(my + o, N_DEV)
            slot = N_DEV - 1 - o
            rdma = pltpu.make_async_remote_copy(
                src_ref=pb_ref.at[pl.ds(tgt * CH, CH), :],
                dst_ref=comm_rs.at[slot],
                send_sem=rs_send.at[o - 1],
                recv_sem=rs_recv.at[slot],
                device_id=(tgt,),
                device_id_type=pl.DeviceIdType.MESH,
            )
            rdma.start()
            rs_rdmas.append(rdma)
        for r in rs_rdmas:
            r.wait_recv()
        acc = out_ref[0, pl.ds(my * CH, CH), :]
        for s in range(N_DEV - 1):
            acc = acc + comm_rs[s].astype(jnp.float32)
        out_ref[0, pl.ds(my * CH, CH), :] = acc
        ag_src[:, :] = acc.astype(jnp.bfloat16)
        for r in rs_rdmas:
            r.wait_send()

        ag_rdmas = []
        for o in range(1, N_DEV):
            tgt = lax.rem(my + o, N_DEV)
            slot = N_DEV - 1 - o
            rdma = pltpu.make_async_remote_copy(
                src_ref=ag_src,
                dst_ref=ag_buf.at[slot],
                send_sem=ag_send.at[o - 1],
                recv_sem=ag_recv.at[slot],
                device_id=(tgt,),
                device_id_type=pl.DeviceIdType.MESH,
            )
            rdma.start()
            ag_rdmas.append(rdma)
        for s, r in enumerate(ag_rdmas):
            r.wait_recv()
        for s in range(N_DEV - 1):
            sender = lax.rem(my + 1 + s, N_DEV)
            out_ref[0, pl.ds(sender * CH, CH), :] = (
                ag_buf[s].astype(jnp.float32))
        for r in ag_rdmas:
            r.wait_send()

    return pl.pallas_call(
        body,
        out_shape=jax.ShapeDtypeStruct((1, SQ, D), jnp.float32),
        in_specs=[pl.BlockSpec(memory_space=pltpu.VMEM)] * 5,
        out_specs=pl.BlockSpec(memory_space=pltpu.VMEM),
        scratch_shapes=[
            pltpu.VMEM((SQ, D), jnp.bfloat16),
            pltpu.VMEM((N_DEV - 1, CH, D), jnp.bfloat16),
            pltpu.VMEM((CH, D), jnp.bfloat16),
            pltpu.VMEM((N_DEV - 1, CH, D), jnp.bfloat16),
            pltpu.SemaphoreType.DMA((N_DEV - 1,)),
            pltpu.SemaphoreType.DMA((N_DEV - 1,)),
            pltpu.SemaphoreType.DMA((N_DEV - 1,)),
            pltpu.SemaphoreType.DMA((N_DEV - 1,)),
        ],
    )(x, Wq, Wo, Wk, Wv)


# device time: 36224 ns/iter; 2.8478x vs baseline; 1.1159x over previous
import jax
import jax.numpy as jnp
from jax import lax
from jax.experimental import pallas as pl
from jax.experimental.pallas import tpu as pltpu

N_DEV = 16
SQ = 256
D = 1024
HEADS = 8
DH = 128
CH = SQ // N_DEV
SCALE = 0.08838834764831843


def kernel(x, Wq, Wo, Wk, Wv):
    def body(x_ref, wq_ref, wo_ref, wk_ref, wv_ref, out_ref,
             comm_rs, rs_send, rs_recv, ag_send, ag_recv):
        my = lax.axis_index("i")

        barrier_sem = pltpu.get_barrier_semaphore()
        for o in range(1, N_DEV):
            pl.semaphore_signal(
                barrier_sem, inc=1,
                device_id=(lax.rem(my + o, N_DEV),),
                device_id_type=pl.DeviceIdType.MESH,
            )
        pl.semaphore_wait(barrier_sem, N_DEV - 1)

        xb = x_ref[0, :, :].astype(jnp.bfloat16)
        q = lax.dot_general(
            xb, wq_ref[:, :].astype(jnp.bfloat16),
            (((1,), (0,)), ((), ())), preferred_element_type=jnp.float32)
        k = lax.dot_general(
            xb, wk_ref[:, :].astype(jnp.bfloat16),
            (((1,), (0,)), ((), ())), preferred_element_type=jnp.float32)
        v = lax.dot_general(
            xb, wv_ref[:, :].astype(jnp.bfloat16),
            (((1,), (0,)), ((), ())), preferred_element_type=jnp.float32)

        partial = jnp.zeros((SQ, D), dtype=jnp.float32)
        for h in range(HEADS):
            sl = slice(h * DH, (h + 1) * DH)
            qh = (q[:, sl] * SCALE).astype(jnp.bfloat16)
            kh = k[:, sl].astype(jnp.bfloat16)
            vh = v[:, sl].astype(jnp.bfloat16)
            s = lax.dot_general(qh, kh, (((1,), (1,)), ((), ())),
                                preferred_element_type=jnp.float32)
            m = jnp.max(s, axis=1, keepdims=True)
            p = jnp.exp(s - m)
            l = jnp.sum(p, axis=1, keepdims=True)
            o_h = lax.dot_general(p.astype(jnp.bfloat16), vh,
                                  (((1,), (0,)), ((), ())),
                                  preferred_element_type=jnp.float32)
            o_h = o_h / l
            partial = partial + lax.dot_general(
                o_h.astype(jnp.bfloat16),
                wo_ref[sl, :].astype(jnp.bfloat16),
                (((1,), (0,)), ((), ())), preferred_element_type=jnp.float32)
        out_ref[0, :, :] = partial.astype(jnp.bfloat16)

        rs_rdmas = []
        for o in range(1, N_DEV):
            tgt = lax.rem(my + o, N_DEV)
            slot = N_DEV - 1 - o
            rdma = pltpu.make_async_remote_copy(
                src_ref=out_ref.at[0, pl.ds(tgt * CH, CH), :],
                dst_ref=comm_rs.at[slot],
                send_sem=rs_send.at[o - 1],
                recv_sem=rs_recv.at[slot],
                device_id=(tgt,),
                device_id_type=pl.DeviceIdType.MESH,
            )
            rdma.start()
            rs_rdmas.append(rdma)
        for r in rs_rdmas:
            r.wait_recv()
        acc = out_ref[0, pl.ds(my * CH, CH), :].astype(jnp.float32)
        for s in range(N_DEV - 1):
            acc = acc + comm_rs[s].astype(jnp.float32)
        out_ref[0, pl.ds(my * CH, CH), :] = acc.astype(jnp.bfloat16)
        for r in rs_rdmas:
            r.wait_send()

        ag_rdmas = []
        for o in range(1, N_DEV):
            tgt = lax.rem(my + o, N_DEV)
            rdma = pltpu.make_async_remote_copy(
                src_ref=out_ref.at[0, pl.ds(my * CH, CH), :],
                dst_ref=out_ref.at[0, pl.ds(my * CH, CH), :],
                send_sem=ag_send.at[o - 1],
                recv_sem=ag_recv.at[o - 1],
                device_id=(tgt,),
                device_id_type=pl.DeviceIdType.MESH,
            )
            rdma.start()
            ag_rdmas.append(rdma)
        for r in ag_rdmas:
            r.wait_recv()
        for r in ag_rdmas:
            r.wait_send()

    return pl.pallas_call(
        body,
        out_shape=jax.ShapeDtypeStruct((1, SQ, D), jnp.bfloat16),
        in_specs=[pl.BlockSpec(memory_space=pltpu.VMEM)] * 5,
        out_specs=pl.BlockSpec(memory_space=pltpu.VMEM),
        scratch_shapes=[
            pltpu.VMEM((N_DEV - 1, CH, D), jnp.bfloat16),
            pltpu.SemaphoreType.DMA((N_DEV - 1,)),
            pltpu.SemaphoreType.DMA((N_DEV - 1,)),
            pltpu.SemaphoreType.DMA((N_DEV - 1,)),
            pltpu.SemaphoreType.DMA((N_DEV - 1,)),
        ],
        compiler_params=pltpu.CompilerParams(collective_id=0),
    )(x, Wq, Wo, Wk, Wv)
